# baseline (speedup 1.0000x reference)
import jax
import jax.numpy as jnp
from jax.experimental import pallas as pl
from jax.experimental.pallas import tpu as pltpu

_CB = 21


def _nll2d_bwd_body(scal_ref, ii_ref, weight_ref, target_ref, out_ref, tnorm_ref):
    cb = pl.program_id(1)
    nclass = pl.num_programs(1) * _CB

    @pl.when(cb == 0)
    def _():
        tgt = target_ref[0]
        tc = jnp.clip(tgt, 0, nclass - 1)
        tnorm_ref[...] = jnp.where(tgt == ii_ref[0], nclass, tc)

    tnorm = tnorm_ref[...]
    for j in range(_CB):
        c = cb * _CB + j
        val = -scal_ref[0] * weight_ref[c]
        out_ref[0, j] = jnp.where(tnorm == c, val, 0.0)


def kernel(grad_output, x, target, weight, reduction, ignore_index, total_weight):
    n_, c_, h_, w_ = x.shape
    assert c_ % _CB == 0
    scal = jnp.where(reduction == 1, grad_output / total_weight, grad_output)
    scal = jnp.asarray(scal, x.dtype).reshape((1,))
    ii = jnp.asarray(ignore_index, jnp.int32).reshape((1,))
    weight = jnp.asarray(weight, x.dtype)

    out = pl.pallas_call(
        _nll2d_bwd_body,
        grid=(n_, c_ // _CB),
        in_specs=[
            pl.BlockSpec(memory_space=pltpu.SMEM),
            pl.BlockSpec(memory_space=pltpu.SMEM),
            pl.BlockSpec(memory_space=pltpu.SMEM),
            pl.BlockSpec((1, h_, w_), lambda n, c: (n, 0, 0)),
        ],
        out_specs=pl.BlockSpec((1, _CB, h_, w_), lambda n, c: (n, c, 0, 0)),
        out_shape=jax.ShapeDtypeStruct((n_, c_, h_, w_), x.dtype),
        scratch_shapes=[pltpu.VMEM((h_, w_), jnp.int32)],
        compiler_params=pltpu.CompilerParams(
            dimension_semantics=("arbitrary", "arbitrary"),
        ),
    )(scal, ii, weight, target)
    return out


# speedup vs baseline: 1.0265x; 1.0265x over previous
import jax
import jax.numpy as jnp
from jax.experimental import pallas as pl
from jax.experimental.pallas import tpu as pltpu

_CB = 7


def _nll2d_bwd_body(scal_ref, ii_ref, weight_ref, target_ref, out_ref, tnorm_ref):
    cb = pl.program_id(1)
    nclass = pl.num_programs(1) * _CB

    @pl.when(cb == 0)
    def _():
        tgt = target_ref[0]
        tc = jnp.clip(tgt, 0, nclass - 1)
        tnorm_ref[...] = jnp.where(tgt == ii_ref[0], nclass, tc)

    tnorm = tnorm_ref[...]
    for j in range(_CB):
        c = cb * _CB + j
        val = -scal_ref[0] * weight_ref[c]
        out_ref[0, j] = jnp.where(tnorm == c, val, 0.0)


def kernel(grad_output, x, target, weight, reduction, ignore_index, total_weight):
    n_, c_, h_, w_ = x.shape
    assert c_ % _CB == 0
    scal = jnp.where(reduction == 1, grad_output / total_weight, grad_output)
    scal = jnp.asarray(scal, x.dtype).reshape((1,))
    ii = jnp.asarray(ignore_index, jnp.int32).reshape((1,))
    weight = jnp.asarray(weight, x.dtype)

    out = pl.pallas_call(
        _nll2d_bwd_body,
        grid=(n_, c_ // _CB),
        in_specs=[
            pl.BlockSpec(memory_space=pltpu.SMEM),
            pl.BlockSpec(memory_space=pltpu.SMEM),
            pl.BlockSpec(memory_space=pltpu.SMEM),
            pl.BlockSpec((1, h_, w_), lambda n, c: (n, 0, 0)),
        ],
        out_specs=pl.BlockSpec((1, _CB, h_, w_), lambda n, c: (n, c, 0, 0)),
        out_shape=jax.ShapeDtypeStruct((n_, c_, h_, w_), x.dtype),
        scratch_shapes=[pltpu.VMEM((h_, w_), jnp.int32)],
        compiler_params=pltpu.CompilerParams(
            dimension_semantics=("parallel", "arbitrary"),
        ),
    )(scal, ii, weight, target)
    return out


# speedup vs baseline: 1.0288x; 1.0022x over previous
import jax
import jax.numpy as jnp
from jax.experimental import pallas as pl
from jax.experimental.pallas import tpu as pltpu

_CB = 7


def _nll2d_bwd_body(scal_ref, ii_ref, weight_ref, target_ref, out_ref):
    cb = pl.program_id(1)
    nclass = pl.num_programs(1) * _CB

    tgt = target_ref[0]
    tc = jnp.clip(tgt, 0, nclass - 1)
    tnorm = jnp.where(tgt == ii_ref[0], nclass, tc)
    for j in range(_CB):
        c = cb * _CB + j
        val = -scal_ref[0] * weight_ref[c]
        out_ref[0, j] = jnp.where(tnorm == c, val, 0.0)


def kernel(grad_output, x, target, weight, reduction, ignore_index, total_weight):
    n_, c_, h_, w_ = x.shape
    assert c_ % _CB == 0
    scal = jnp.where(reduction == 1, grad_output / total_weight, grad_output)
    scal = jnp.asarray(scal, x.dtype).reshape((1,))
    ii = jnp.asarray(ignore_index, jnp.int32).reshape((1,))
    weight = jnp.asarray(weight, x.dtype)

    out = pl.pallas_call(
        _nll2d_bwd_body,
        grid=(n_, c_ // _CB),
        in_specs=[
            pl.BlockSpec(memory_space=pltpu.SMEM),
            pl.BlockSpec(memory_space=pltpu.SMEM),
            pl.BlockSpec(memory_space=pltpu.SMEM),
            pl.BlockSpec((1, h_, w_), lambda n, c: (n, 0, 0)),
        ],
        out_specs=pl.BlockSpec((1, _CB, h_, w_), lambda n, c: (n, c, 0, 0)),
        out_shape=jax.ShapeDtypeStruct((n_, c_, h_, w_), x.dtype),
        compiler_params=pltpu.CompilerParams(
            dimension_semantics=("parallel", "arbitrary"),
        ),
    )(scal, ii, weight, target)
    return out
